# baseline (speedup 1.0000x reference)
import functools

import jax
import jax.numpy as jnp
from jax import lax
from jax.experimental import pallas as pl
from jax.experimental.pallas import tpu as pltpu
from jax.experimental.pallas import tpu_sc as plsc

V = 1_000_000
D = 64
B = 4096
L = 50

NC = 2
NS = 16
LANES = 16
NW = NC * NS
QPW = B // NW
QC = 16
NCHUNK = QPW // QC
RPC = QC * L
GSZ = 80
NG = RPC // GSZ
ND = D // LANES
RW = 128
TAIL = L - 48

_GDN = lax.GatherDimensionNumbers(
    offset_dims=(), collapsed_slice_dims=(0,), start_index_map=(0,))


def _bcast_lane(vec, lane):
    idx = jnp.full((LANES, 1), lane, jnp.int32)
    return lax.gather(vec, idx, dimension_numbers=_GDN, slice_sizes=(1,),
                      mode=lax.GatherScatterMode.PROMISE_IN_BOUNDS)


def _perm(vec, idx):
    return lax.gather(vec, idx[:, None], dimension_numbers=_GDN,
                      slice_sizes=(1,),
                      mode=lax.GatherScatterMode.PROMISE_IN_BOUNDS)


def _all_reduce(vec, op, iota):
    for k in (8, 4, 2, 1):
        vec = op(vec, _perm(vec, iota ^ k))
    return vec


def _sc_body(query_hbm, table_hbm, weights_hbm, bias_hbm, out_hbm,
             idx_v, w_v, p_v, rows_v, out_v, bias_v, sem_r, sem_w):
    wid = lax.axis_index("s") * NC + lax.axis_index("c")
    iota = lax.iota(jnp.int32, LANES)
    tail_mask = iota < TAIL

    pltpu.sync_copy(query_hbm.at[pl.ds(wid * QPW * L, QPW * L)], idx_v)
    pltpu.sync_copy(bias_hbm, bias_v)
    bias_regs = [bias_v[pl.ds(d * LANES, LANES)] for d in range(ND)]

    for g in range(NCHUNK):
        row0 = g * RPC
        rcopies = [pltpu.async_copy(
            table_hbm.at[idx_v.at[pl.ds(row0 + j * GSZ, GSZ)]],
            rows_v.at[pl.ds(j * GSZ, GSZ)], sem_r) for j in range(NG)]
        wcopies = [pltpu.async_copy(
            weights_hbm.at[idx_v.at[pl.ds(row0 + j * GSZ, GSZ)]],
            w_v.at[pl.ds(j * GSZ, GSZ)], sem_w) for j in range(NG)]
        for c in wcopies:
            c.wait()

        def softmax_q(q, _):
            off = q * L
            c0 = w_v[pl.ds(off, LANES)]
            c1 = w_v[pl.ds(off + 16, LANES)]
            c2 = w_v[pl.ds(off + 32, LANES)]
            c3 = w_v[pl.ds(off + 48, LANES)]
            c3m = jnp.where(tail_mask, c3, jnp.float32(-1e30))
            m = _all_reduce(jnp.maximum(jnp.maximum(c0, c1),
                                        jnp.maximum(c2, c3m)),
                            jnp.maximum, iota)
            e0 = jnp.exp(c0 - m)
            e1 = jnp.exp(c1 - m)
            e2 = jnp.exp(c2 - m)
            e3 = jnp.where(tail_mask, jnp.exp(c3 - m), 0.0)
            sinv = 1.0 / _all_reduce(e0 + e1 + e2 + e3, jnp.add, iota)
            p0 = q * D
            p_v[pl.ds(p0, LANES)] = e0 * sinv
            p_v[pl.ds(p0 + 16, LANES)] = e1 * sinv
            p_v[pl.ds(p0 + 32, LANES)] = e2 * sinv
            p_v[pl.ds(p0 + 48, LANES)] = e3 * sinv
            return 0

        lax.fori_loop(0, QC, softmax_q, 0)

        for c in rcopies:
            c.wait()

        def acc_q(q, _):
            row_q = q * L
            accs = [jnp.zeros((LANES,), jnp.float32) for _ in range(ND)]
            for tc in range(4):
                pc = p_v[pl.ds(q * D + tc * LANES, LANES)]

                def tbody(tt, accs, tc=tc, pc=pc):
                    pb = _bcast_lane(pc, tt)
                    ridx = row_q + tc * LANES + tt
                    return tuple(
                        accs[d] + pb * rows_v[ridx, pl.ds(d * LANES, LANES)]
                        for d in range(ND))

                n_t = TAIL if tc == 3 else LANES
                accs = lax.fori_loop(0, n_t, tbody, tuple(accs))
            a0, a1, a2, a3 = accs
            s2v = jnp.maximum(
                _all_reduce(a0 * a0 + a1 * a1 + a2 * a2 + a3 * a3,
                            jnp.add, iota),
                jnp.float32(1e-35))
            bits = lax.bitcast_convert_type(s2v, jnp.int32)
            y = lax.bitcast_convert_type(
                jnp.int32(0x5F3759DF) - lax.shift_right_logical(bits, 1),
                jnp.float32)
            y = y * (1.5 - 0.5 * s2v * y * y)
            y = y * (1.5 - 0.5 * s2v * y * y)
            y = y * (1.5 - 0.5 * s2v * y * y)
            invn = 1.0 / (s2v * y + 1e-4)
            for d in range(ND):
                out_v[q, pl.ds(d * LANES, LANES)] = (
                    accs[d] * invn + bias_regs[d])
            return 0

        lax.fori_loop(0, QC, acc_q, 0)
        pltpu.sync_copy(out_v, out_hbm.at[pl.ds(wid * QPW + g * QC, QC)])


@functools.partial(jax.jit)
def _encode(query_flat, table, weights_flat, bias):
    mesh = plsc.VectorSubcoreMesh(core_axis_name="c", subcore_axis_name="s")
    run = functools.partial(
        pl.kernel,
        out_type=jax.ShapeDtypeStruct((B, D), jnp.float32),
        mesh=mesh,
        compiler_params=pltpu.CompilerParams(use_tc_tiling_on_sc=False),
        scratch_types=[
            pltpu.VMEM((QPW * L,), jnp.int32),
            pltpu.VMEM((RPC + LANES,), jnp.float32),
            pltpu.VMEM((QC * D,), jnp.float32),
            pltpu.VMEM((RPC, D), jnp.float32),
            pltpu.VMEM((QC, D), jnp.float32),
            pltpu.VMEM((D,), jnp.float32),
            pltpu.SemaphoreType.DMA,
            pltpu.SemaphoreType.DMA,
        ],
    )(_sc_body)
    return run(query_flat, table, weights_flat, bias)


def kernel(query, table, weights, bias):
    query_flat = query.reshape(-1).astype(jnp.int32)
    return _encode(query_flat, table, weights.reshape(-1),
                   bias.astype(jnp.float32))
